# baseline (speedup 1.0000x reference)
import functools

import jax
import jax.numpy as jnp
from jax import lax
from jax.experimental import pallas as pl
from jax.experimental.pallas import tpu as pltpu
from jax.experimental.pallas import tpu_sc as plsc

NC, NS, LANES = 2, 16, 16
NROW = 10000
NP = 10240
NEDGE = 320000
CHUNK = 128
NCHUNK = NEDGE // CHUNK
CPT = -(-NCHUNK // (NC * NS))
CPT2 = -(-NCHUNK // NS)
D = 128
DH = D // 2
BLK = 10000
NB = 5
PRE = 4


def _sc_deg_body(edges, o0, o1, idxc, buf, acc, sem):
    c = lax.axis_index("c")
    s = lax.axis_index("s")
    tid = c * NS + s
    tile_base = NCHUNK + tid * CPT
    last = NCHUNK - (NC * NS - 1) * CPT

    def fill(val):
        def body(i, _):
            buf[i] = jnp.full((LANES,), val, jnp.float32)
            return 0
        lax.fori_loop(0, CHUNK, body, 0)

    fill(0.0)
    for k in range(NP // NS // CHUNK):
        pltpu.sync_copy(buf, acc.at[pl.ds(s * (NP // NS) + k * CHUNK, CHUNK)])
    fill(1.0)

    @pl.when(tid < NC * NS - 1)
    def _stage_full():
        pltpu.sync_copy(edges.at[pl.ds(tile_base, CPT)], idxc.at[pl.ds(0, CPT)])

    @pl.when(tid == NC * NS - 1)
    def _stage_last():
        pltpu.sync_copy(edges.at[pl.ds(tile_base, last)], idxc.at[pl.ds(0, last)])

    plsc.subcore_barrier()

    def fire(t, _):
        @pl.when(tid * CPT + t < NCHUNK)
        def _():
            pltpu.async_copy(buf, acc.at[idxc.at[t]], sem, add=True)
        return 0
    lax.fori_loop(0, CPT, fire, 0)

    def drain(t, _):
        @pl.when(tid * CPT + t < NCHUNK)
        def _():
            pltpu.make_async_copy(buf, acc.at[idxc.at[0]], sem).wait()
        return 0
    lax.fori_loop(0, CPT, drain, 0)

    plsc.subcore_barrier()
    nrt = NP // NS

    @pl.when(c == 0)
    def _out0():
        pltpu.sync_copy(acc.at[pl.ds(s * nrt, nrt)], o0.at[pl.ds(s * nrt, nrt)])

    @pl.when(c == 1)
    def _out1():
        pltpu.sync_copy(acc.at[pl.ds(s * nrt, nrt)], o1.at[pl.ds(s * nrt, nrt)])


def _sc_agg_body(y0, y1, rowi, coli, o0, o1, idxr, idxc,
                 b0, b1, b2, b3, b4, acc,
                 g0, g1, g2, g3, g4, s0, s1, s2, s3, s4):
    bufs = (b0, b1, b2, b3, b4)
    gsem = (g0, g1, g2, g3, g4)
    ssem = (s0, s1, s2, s3, s4)
    c = lax.axis_index("c")
    s = lax.axis_index("s")
    tile_base = s * CPT2
    last = NCHUNK - (NS - 1) * CPT2
    ec = jnp.where(s == NS - 1, last, CPT2)
    nrt = NP // NS

    def zrow(i, _):
        def zcol(j, _):
            b0[i, pl.ds(j * LANES, LANES)] = jnp.zeros((LANES,), jnp.float32)
            return 0
        return lax.fori_loop(0, DH // LANES, zcol, 0)
    lax.fori_loop(0, CHUNK, zrow, 0)
    for k in range(nrt // CHUNK):
        pltpu.async_copy(b0, acc.at[pl.ds(s * nrt + k * CHUNK, CHUNK)], s0)
    for k in range(nrt // CHUNK):
        pltpu.make_async_copy(b0, acc.at[pl.ds(s * nrt, CHUNK)], s0).wait()

    @pl.when(s < NS - 1)
    def _stage_full():
        pltpu.sync_copy(rowi.at[pl.ds(tile_base, CPT2)],
                        idxr.at[pl.ds(0, CPT2)])
        pltpu.sync_copy(coli.at[pl.ds(tile_base, CPT2)],
                        idxc.at[pl.ds(0, CPT2)])

    @pl.when(s == NS - 1)
    def _stage_last():
        pltpu.sync_copy(rowi.at[pl.ds(tile_base, last)],
                        idxr.at[pl.ds(0, last)])
        pltpu.sync_copy(coli.at[pl.ds(tile_base, last)],
                        idxc.at[pl.ds(0, last)])

    plsc.subcore_barrier()

    def run(tab):
        for j in range(PRE):
            pltpu.async_copy(tab.at[idxr.at[j]], bufs[j], gsem[j])

        def group(gi, _):
            for u in range(NB):
                tt = gi * NB + u
                sj = (u + PRE) % NB

                @pl.when(tt + PRE < ec)
                def _fire():
                    @pl.when(tt >= NB - PRE)
                    def _recycle():
                        pltpu.make_async_copy(
                            bufs[sj], acc.at[idxc.at[0]], ssem[sj]).wait()
                    pltpu.async_copy(tab.at[idxr.at[tt + PRE]], bufs[sj],
                                     gsem[sj])

                @pl.when(tt < ec)
                def _consume():
                    pltpu.make_async_copy(tab.at[idxr.at[0]], bufs[u],
                                          gsem[u]).wait()
                    pltpu.async_copy(bufs[u], acc.at[idxc.at[tt]], ssem[u],
                                     add=True)
            return 0
        lax.fori_loop(0, -(-CPT2 // NB), group, 0)

        for u in range(NB):
            pltpu.make_async_copy(bufs[u], acc.at[idxc.at[0]], ssem[u]).wait()

    @pl.when(c == 0)
    def _run0():
        run(y0)

    @pl.when(c == 1)
    def _run1():
        run(y1)

    plsc.subcore_barrier()

    @pl.when(c == 0)
    def _out0():
        pltpu.sync_copy(acc.at[pl.ds(s * nrt, nrt)], o0.at[pl.ds(s * nrt, nrt)])

    @pl.when(c == 1)
    def _out1():
        pltpu.sync_copy(acc.at[pl.ds(s * nrt, nrt)], o1.at[pl.ds(s * nrt, nrt)])


@functools.lru_cache(maxsize=None)
def _sc_kernels():
    mesh = plsc.VectorSubcoreMesh(
        core_axis_name="c", subcore_axis_name="s", num_cores=NC, num_subcores=NS)
    sc_deg = pl.kernel(
        _sc_deg_body,
        out_type=[jax.ShapeDtypeStruct((NP, 16), jnp.float32),
                  jax.ShapeDtypeStruct((NP, 16), jnp.float32)],
        mesh=mesh,
        scratch_types=[
            pltpu.VMEM((CPT, CHUNK), jnp.int32),
            pltpu.VMEM((CHUNK, 16), jnp.float32),
            pltpu.VMEM_SHARED((NP, 16), jnp.float32),
            pltpu.SemaphoreType.DMA,
        ],
        compiler_params=pltpu.CompilerParams(use_tc_tiling_on_sc=False))
    sc_agg = pl.kernel(
        _sc_agg_body,
        out_type=[jax.ShapeDtypeStruct((NP, DH), jnp.float32),
                  jax.ShapeDtypeStruct((NP, DH), jnp.float32)],
        mesh=mesh,
        scratch_types=[
            pltpu.VMEM((CPT2, CHUNK), jnp.int32),
            pltpu.VMEM((CPT2, CHUNK), jnp.int32),
        ] + [pltpu.VMEM((CHUNK, DH), jnp.float32)] * NB + [
            pltpu.VMEM_SHARED((NP, DH), jnp.float32),
        ] + [pltpu.SemaphoreType.DMA] * (2 * NB),
        compiler_params=pltpu.CompilerParams(use_tc_tiling_on_sc=False))
    return sc_deg, sc_agg


def _dinv_of(d0, d1):
    deg = d0[:, :1] + d1[:, :1] + 1.0
    return lax.rsqrt(deg)


def _k1_body(x, d0, d1, wg, wl1, bl1, wl2, bl2, ylo, yhi, hlin):
    dinv = _dinv_of(d0, d1)
    xv = x[...]
    y = dinv * jnp.dot(xv, wg[...], preferred_element_type=jnp.float32)
    ylo[...] = y[:, :DH]
    yhi[...] = y[:, DH:]
    t = jnp.maximum(
        jnp.dot(xv, wl1[...], preferred_element_type=jnp.float32) + bl1[...], 0.0)
    hlin[...] = jnp.dot(t, wl2[...], preferred_element_type=jnp.float32) + bl2[...]


def _k3_body(alo, ahi, ylo, yhi, d0, d1, w, b, y2lo, y2hi):
    dinv = _dinv_of(d0, d1)
    h = jnp.concatenate([alo[...] + ylo[...], ahi[...] + yhi[...]], axis=1)
    h1 = jnp.maximum(dinv * h + b[...], 0.0)
    y2 = dinv * jnp.dot(h1, w[...], preferred_element_type=jnp.float32)
    y2lo[...] = y2[:, :DH]
    y2hi[...] = y2[:, DH:]


def _k5_body(alo, ahi, ylo, yhi, d0, d1, b2, hlin, wf, bf, logits, probs):
    dinv = _dinv_of(d0, d1)
    h = jnp.concatenate([alo[...] + ylo[...], ahi[...] + yhi[...]], axis=1)
    hg = jnp.maximum(dinv * h + b2[...], 0.0)
    hl = jnp.maximum(hlin[...], 0.0)
    wf_v = wf[...]
    lg = (jnp.dot(hg, wf_v[:D], preferred_element_type=jnp.float32)
          + jnp.dot(hl, wf_v[D:], preferred_element_type=jnp.float32) + bf[...])
    m = jnp.max(lg, axis=1, keepdims=True)
    e = jnp.exp(lg - m)
    logits[...] = lg
    probs[...] = e / jnp.sum(e, axis=1, keepdims=True)


def _row_spec(cols):
    return pl.BlockSpec((BLK, cols), lambda i: (i, 0))


def _full_spec(r, c):
    return pl.BlockSpec((r, c), lambda i: (0, 0))


_GRID = NROW // BLK

_k1 = pl.pallas_call(
    _k1_body, grid=(_GRID,),
    in_specs=[_row_spec(D), _row_spec(16), _row_spec(16), _full_spec(D, D),
              _full_spec(D, D), _full_spec(1, D), _full_spec(D, D),
              _full_spec(1, D)],
    out_specs=[_row_spec(DH), _row_spec(DH), _row_spec(D)],
    out_shape=[jax.ShapeDtypeStruct((NROW, DH), jnp.float32),
               jax.ShapeDtypeStruct((NROW, DH), jnp.float32),
               jax.ShapeDtypeStruct((NROW, D), jnp.float32)])

_k3 = pl.pallas_call(
    _k3_body, grid=(_GRID,),
    in_specs=[_row_spec(DH), _row_spec(DH), _row_spec(DH), _row_spec(DH),
              _row_spec(16), _row_spec(16), _full_spec(D, D), _full_spec(1, D)],
    out_specs=[_row_spec(DH), _row_spec(DH)],
    out_shape=[jax.ShapeDtypeStruct((NROW, DH), jnp.float32),
               jax.ShapeDtypeStruct((NROW, DH), jnp.float32)])

_k5 = pl.pallas_call(
    _k5_body, grid=(_GRID,),
    in_specs=[_row_spec(DH), _row_spec(DH), _row_spec(DH), _row_spec(DH),
              _row_spec(16), _row_spec(16), _full_spec(1, D), _row_spec(D),
              _full_spec(2 * D, 40), _full_spec(1, 40)],
    out_specs=[_row_spec(40), _row_spec(40)],
    out_shape=[jax.ShapeDtypeStruct((NROW, 40), jnp.float32),
               jax.ShapeDtypeStruct((NROW, 40), jnp.float32)])


def kernel(x, edge_index, Wg1, bg1, Wg2, bg2, Wl1, bl1, Wl2, bl2, Wf, bf):
    edges = edge_index.reshape(2 * NCHUNK, CHUNK)
    rowp = edges[:NCHUNK]
    colp = edges[NCHUNK:]

    sc_deg, sc_agg = _sc_kernels()
    d0, d1 = sc_deg(edges)

    y1lo, y1hi, hlin = _k1(x, d0, d1, Wg1, Wl1, bl1.reshape(1, D), Wl2,
                           bl2.reshape(1, D))
    a1lo, a1hi = sc_agg(y1lo, y1hi, rowp, colp)
    y2lo, y2hi = _k3(a1lo, a1hi, y1lo, y1hi, d0, d1, Wg2, bg1.reshape(1, D))
    a2lo, a2hi = sc_agg(y2lo, y2hi, rowp, colp)
    logits, probs = _k5(a2lo, a2hi, y2lo, y2hi, d0, d1, bg2.reshape(1, D),
                        hlin, Wf, bf.reshape(1, 40))
    return logits, probs


# speedup vs baseline: 1.0170x; 1.0170x over previous
import functools

import jax
import jax.numpy as jnp
from jax import lax
from jax.experimental import pallas as pl
from jax.experimental.pallas import tpu as pltpu
from jax.experimental.pallas import tpu_sc as plsc

NC, NS, LANES = 2, 16, 16
NROW = 10000
NP = 10240
NEDGE = 320000
CHUNK = 128
NCHUNK = NEDGE // CHUNK
CPT = -(-NCHUNK // (NC * NS))
CPT2 = -(-NCHUNK // NS)
D = 128
DH = D // 2
BLK = 5000
NB = 5
PRE = 4


def _sc_deg_body(edges, o0, o1, idxc, buf, acc, sem):
    c = lax.axis_index("c")
    s = lax.axis_index("s")
    tid = c * NS + s
    tile_base = NCHUNK + tid * CPT
    last = NCHUNK - (NC * NS - 1) * CPT

    def fill(val):
        def body(i, _):
            buf[i] = jnp.full((LANES,), val, jnp.float32)
            return 0
        lax.fori_loop(0, CHUNK, body, 0)

    fill(0.0)
    for k in range(NP // NS // CHUNK):
        pltpu.sync_copy(buf, acc.at[pl.ds(s * (NP // NS) + k * CHUNK, CHUNK)])
    fill(1.0)

    @pl.when(tid < NC * NS - 1)
    def _stage_full():
        pltpu.sync_copy(edges.at[pl.ds(tile_base, CPT)], idxc.at[pl.ds(0, CPT)])

    @pl.when(tid == NC * NS - 1)
    def _stage_last():
        pltpu.sync_copy(edges.at[pl.ds(tile_base, last)], idxc.at[pl.ds(0, last)])

    plsc.subcore_barrier()

    def fire(t, _):
        @pl.when(tid * CPT + t < NCHUNK)
        def _():
            pltpu.async_copy(buf, acc.at[idxc.at[t]], sem, add=True)
        return 0
    lax.fori_loop(0, CPT, fire, 0)

    def drain(t, _):
        @pl.when(tid * CPT + t < NCHUNK)
        def _():
            pltpu.make_async_copy(buf, acc.at[idxc.at[0]], sem).wait()
        return 0
    lax.fori_loop(0, CPT, drain, 0)

    plsc.subcore_barrier()
    nrt = NP // NS

    @pl.when(c == 0)
    def _out0():
        pltpu.sync_copy(acc.at[pl.ds(s * nrt, nrt)], o0.at[pl.ds(s * nrt, nrt)])

    @pl.when(c == 1)
    def _out1():
        pltpu.sync_copy(acc.at[pl.ds(s * nrt, nrt)], o1.at[pl.ds(s * nrt, nrt)])


def _sc_agg_body(y0, y1, rowi, coli, o0, o1, idxr, idxc,
                 b0, b1, b2, b3, b4, acc,
                 g0, g1, g2, g3, g4, s0, s1, s2, s3, s4):
    bufs = (b0, b1, b2, b3, b4)
    gsem = (g0, g1, g2, g3, g4)
    ssem = (s0, s1, s2, s3, s4)
    c = lax.axis_index("c")
    s = lax.axis_index("s")
    tile_base = s * CPT2
    last = NCHUNK - (NS - 1) * CPT2
    ec = jnp.where(s == NS - 1, last, CPT2)
    nrt = NP // NS

    def zrow(i, _):
        def zcol(j, _):
            b0[i, pl.ds(j * LANES, LANES)] = jnp.zeros((LANES,), jnp.float32)
            return 0
        return lax.fori_loop(0, DH // LANES, zcol, 0)
    lax.fori_loop(0, CHUNK, zrow, 0)
    for k in range(nrt // CHUNK):
        pltpu.async_copy(b0, acc.at[pl.ds(s * nrt + k * CHUNK, CHUNK)], s0)
    for k in range(nrt // CHUNK):
        pltpu.make_async_copy(b0, acc.at[pl.ds(s * nrt, CHUNK)], s0).wait()

    @pl.when(s < NS - 1)
    def _stage_full():
        pltpu.sync_copy(rowi.at[pl.ds(tile_base, CPT2)],
                        idxr.at[pl.ds(0, CPT2)])
        pltpu.sync_copy(coli.at[pl.ds(tile_base, CPT2)],
                        idxc.at[pl.ds(0, CPT2)])

    @pl.when(s == NS - 1)
    def _stage_last():
        pltpu.sync_copy(rowi.at[pl.ds(tile_base, last)],
                        idxr.at[pl.ds(0, last)])
        pltpu.sync_copy(coli.at[pl.ds(tile_base, last)],
                        idxc.at[pl.ds(0, last)])

    plsc.subcore_barrier()

    def run(tab):
        for j in range(PRE):
            pltpu.async_copy(tab.at[idxr.at[j]], bufs[j], gsem[j])

        def group(gi, _):
            for u in range(NB):
                tt = gi * NB + u
                sj = (u + PRE) % NB

                @pl.when(tt + PRE < ec)
                def _fire():
                    @pl.when(tt >= NB - PRE)
                    def _recycle():
                        pltpu.make_async_copy(
                            bufs[sj], acc.at[idxc.at[0]], ssem[sj]).wait()
                    pltpu.async_copy(tab.at[idxr.at[tt + PRE]], bufs[sj],
                                     gsem[sj])

                @pl.when(tt < ec)
                def _consume():
                    pltpu.make_async_copy(tab.at[idxr.at[0]], bufs[u],
                                          gsem[u]).wait()
                    pltpu.async_copy(bufs[u], acc.at[idxc.at[tt]], ssem[u],
                                     add=True)
            return 0
        lax.fori_loop(0, -(-CPT2 // NB), group, 0)

        for u in range(NB):
            pltpu.make_async_copy(bufs[u], acc.at[idxc.at[0]], ssem[u]).wait()

    @pl.when(c == 0)
    def _run0():
        run(y0)

    @pl.when(c == 1)
    def _run1():
        run(y1)

    plsc.subcore_barrier()

    @pl.when(c == 0)
    def _out0():
        pltpu.sync_copy(acc.at[pl.ds(s * nrt, nrt)], o0.at[pl.ds(s * nrt, nrt)])

    @pl.when(c == 1)
    def _out1():
        pltpu.sync_copy(acc.at[pl.ds(s * nrt, nrt)], o1.at[pl.ds(s * nrt, nrt)])


@functools.lru_cache(maxsize=None)
def _sc_kernels():
    mesh = plsc.VectorSubcoreMesh(
        core_axis_name="c", subcore_axis_name="s", num_cores=NC, num_subcores=NS)
    sc_deg = pl.kernel(
        _sc_deg_body,
        out_type=[jax.ShapeDtypeStruct((NP, 16), jnp.float32),
                  jax.ShapeDtypeStruct((NP, 16), jnp.float32)],
        mesh=mesh,
        scratch_types=[
            pltpu.VMEM((CPT, CHUNK), jnp.int32),
            pltpu.VMEM((CHUNK, 16), jnp.float32),
            pltpu.VMEM_SHARED((NP, 16), jnp.float32),
            pltpu.SemaphoreType.DMA,
        ],
        compiler_params=pltpu.CompilerParams(use_tc_tiling_on_sc=False))
    sc_agg = pl.kernel(
        _sc_agg_body,
        out_type=[jax.ShapeDtypeStruct((NP, DH), jnp.float32),
                  jax.ShapeDtypeStruct((NP, DH), jnp.float32)],
        mesh=mesh,
        scratch_types=[
            pltpu.VMEM((CPT2, CHUNK), jnp.int32),
            pltpu.VMEM((CPT2, CHUNK), jnp.int32),
        ] + [pltpu.VMEM((CHUNK, DH), jnp.float32)] * NB + [
            pltpu.VMEM_SHARED((NP, DH), jnp.float32),
        ] + [pltpu.SemaphoreType.DMA] * (2 * NB),
        compiler_params=pltpu.CompilerParams(use_tc_tiling_on_sc=False))
    return sc_deg, sc_agg


def _dinv_of(d0, d1):
    deg = d0[:, :1] + d1[:, :1] + 1.0
    return lax.rsqrt(deg)


def _k1_body(x, d0, d1, wg, wl1, bl1, wl2, bl2, ylo, yhi, hlin):
    dinv = _dinv_of(d0, d1)
    xv = x[...]
    y = dinv * jnp.dot(xv, wg[...], preferred_element_type=jnp.float32)
    ylo[...] = y[:, :DH]
    yhi[...] = y[:, DH:]
    t = jnp.maximum(
        jnp.dot(xv, wl1[...], preferred_element_type=jnp.float32) + bl1[...], 0.0)
    hlin[...] = jnp.dot(t, wl2[...], preferred_element_type=jnp.float32) + bl2[...]


def _k3_body(alo, ahi, ylo, yhi, d0, d1, w, b, y2lo, y2hi):
    dinv = _dinv_of(d0, d1)
    h = jnp.concatenate([alo[...] + ylo[...], ahi[...] + yhi[...]], axis=1)
    h1 = jnp.maximum(dinv * h + b[...], 0.0)
    y2 = dinv * jnp.dot(h1, w[...], preferred_element_type=jnp.float32)
    y2lo[...] = y2[:, :DH]
    y2hi[...] = y2[:, DH:]


def _k5_body(alo, ahi, ylo, yhi, d0, d1, b2, hlin, wf, bf, logits, probs):
    dinv = _dinv_of(d0, d1)
    h = jnp.concatenate([alo[...] + ylo[...], ahi[...] + yhi[...]], axis=1)
    hg = jnp.maximum(dinv * h + b2[...], 0.0)
    hl = jnp.maximum(hlin[...], 0.0)
    wf_v = wf[...]
    lg = (jnp.dot(hg, wf_v[:D], preferred_element_type=jnp.float32)
          + jnp.dot(hl, wf_v[D:], preferred_element_type=jnp.float32) + bf[...])
    m = jnp.max(lg, axis=1, keepdims=True)
    e = jnp.exp(lg - m)
    logits[...] = lg
    probs[...] = e / jnp.sum(e, axis=1, keepdims=True)


def _row_spec(cols):
    return pl.BlockSpec((BLK, cols), lambda i: (i, 0))


def _full_spec(r, c):
    return pl.BlockSpec((r, c), lambda i: (0, 0))


_GRID = NROW // BLK

_k1 = pl.pallas_call(
    _k1_body, grid=(_GRID,),
    in_specs=[_row_spec(D), _row_spec(16), _row_spec(16), _full_spec(D, D),
              _full_spec(D, D), _full_spec(1, D), _full_spec(D, D),
              _full_spec(1, D)],
    out_specs=[_row_spec(DH), _row_spec(DH), _row_spec(D)],
    out_shape=[jax.ShapeDtypeStruct((NROW, DH), jnp.float32),
               jax.ShapeDtypeStruct((NROW, DH), jnp.float32),
               jax.ShapeDtypeStruct((NROW, D), jnp.float32)])

_k3 = pl.pallas_call(
    _k3_body, grid=(_GRID,),
    in_specs=[_row_spec(DH), _row_spec(DH), _row_spec(DH), _row_spec(DH),
              _row_spec(16), _row_spec(16), _full_spec(D, D), _full_spec(1, D)],
    out_specs=[_row_spec(DH), _row_spec(DH)],
    out_shape=[jax.ShapeDtypeStruct((NROW, DH), jnp.float32),
               jax.ShapeDtypeStruct((NROW, DH), jnp.float32)])

_k5 = pl.pallas_call(
    _k5_body, grid=(_GRID,),
    in_specs=[_row_spec(DH), _row_spec(DH), _row_spec(DH), _row_spec(DH),
              _row_spec(16), _row_spec(16), _full_spec(1, D), _row_spec(D),
              _full_spec(2 * D, 40), _full_spec(1, 40)],
    out_specs=[_row_spec(40), _row_spec(40)],
    out_shape=[jax.ShapeDtypeStruct((NROW, 40), jnp.float32),
               jax.ShapeDtypeStruct((NROW, 40), jnp.float32)])


def kernel(x, edge_index, Wg1, bg1, Wg2, bg2, Wl1, bl1, Wl2, bl2, Wf, bf):
    edges = edge_index.reshape(2 * NCHUNK, CHUNK)
    rowp = edges[:NCHUNK]
    colp = edges[NCHUNK:]

    sc_deg, sc_agg = _sc_kernels()
    d0, d1 = sc_deg(edges)

    y1lo, y1hi, hlin = _k1(x, d0, d1, Wg1, Wl1, bl1.reshape(1, D), Wl2,
                           bl2.reshape(1, D))
    a1lo, a1hi = sc_agg(y1lo, y1hi, rowp, colp)
    y2lo, y2hi = _k3(a1lo, a1hi, y1lo, y1hi, d0, d1, Wg2, bg1.reshape(1, D))
    a2lo, a2hi = sc_agg(y2lo, y2hi, rowp, colp)
    logits, probs = _k5(a2lo, a2hi, y2lo, y2hi, d0, d1, bg2.reshape(1, D),
                        hlin, Wf, bf.reshape(1, 40))
    return logits, probs
